# baseline (speedup 1.0000x reference)
import jax
import jax.numpy as jnp
from jax.experimental import pallas as pl
from jax.experimental.pallas import tpu as pltpu

B, T, C = 1, 2048, 768
H, D = 12, 64
K, P = 4, 2
ALPHA, BETA = 2.0, 32.0

TQ = 256
VP = 128


LOG2E = 1.4426950408889634


def _digit_kernel(x_ref, weq_ref, wek_ref, aq_ref, bkT_hi_ref, bkT_lo_ref):
    x = x_ref[...]
    scale = jnp.float32(BETA * LOG2E * (P - 1))
    hb = jnp.float32(BETA * LOG2E / 2)
    dq = jax.nn.sigmoid(
        jnp.dot(x, weq_ref[...], preferred_element_type=jnp.float32)) * scale
    dk = jax.nn.sigmoid(
        jnp.dot(x, wek_ref[...], preferred_element_type=jnp.float32)) * scale
    for h in range(H):
        aq_ref[h] = dq[:, h * K:(h + 1) * K]
        dkT = dk[:, h * K:(h + 1) * K].T
        bkT_hi_ref[h] = dkT + hb
        bkT_lo_ref[h] = dkT - hb


def _v_kernel(x_ref, wvT_ref, v_ref):
    x = x_ref[...]
    vh = jnp.dot(x, wvT_ref[0], preferred_element_type=jnp.float32)
    v_ref[0] = jnp.concatenate(
        [vh, jnp.ones((T, 1), jnp.float32), jnp.zeros((T, VP - D - 1), jnp.float32)],
        axis=1)


def _attn_kernel(aq_ref, bkT_hi_ref, bkT_lo_ref, v_ref, mask_ref,
                 woT_ref, y_ref):
    i = pl.program_id(0)
    one = jnp.float32(1.0)
    outs = []
    for h in range(H):
        aq = aq_ref[h]
        abc = [jnp.broadcast_to(aq[:, l:l + 1], (TQ, TQ)) for l in range(K)]

        def wblock(j, abc=abc, h=h):
            bhi = bkT_hi_ref[h, :, pl.ds(j * TQ, TQ)]
            blo = bkT_lo_ref[h, :, pl.ds(j * TQ, TQ)]
            e = []
            for l in range(K):
                zl = jnp.exp2(jnp.maximum(abc[l] - bhi[l:l + 1, :],
                                          blo[l:l + 1, :] - abc[l]))
                e.append(one + zl)
            s1 = e[3] * e[2]
            s0 = s1 * e[1]
            num = one + e[3] + s1 + s0
            den = e[0] * s0
            return jnp.exp2(num * pl.reciprocal(den, approx=True))

        def body(j, acc, h=h, wblock=wblock):
            vblk = v_ref[h, pl.ds(j * TQ, TQ), :]
            w = wblock(j)
            return acc + jnp.dot(w, vblk, preferred_element_type=jnp.float32)

        acc0 = jnp.zeros((TQ, VP), jnp.float32)
        acc = jax.lax.fori_loop(0, i, body, acc0)

        vblk = v_ref[h, pl.ds(i * TQ, TQ), :]
        w = wblock(i) * mask_ref[...]
        acc = acc + jnp.dot(w, vblk, preferred_element_type=jnp.float32)

        outs.append(acc[:, :D] *
                    pl.reciprocal(acc[:, D:D + 1], approx=True))

    outcat = jnp.concatenate(outs, axis=1)
    y_ref[...] = jnp.dot(outcat, woT_ref[...],
                         preferred_element_type=jnp.float32)


@jax.jit
def _forward(x, Wq, Wk, Wv, Wo, Wdq, Wdk):
    x2 = x.reshape(T, C)
    weq = jnp.einsum('chd,kd->chk', Wq.T.reshape(C, H, D), Wdq).reshape(C, H * K)
    wek = jnp.einsum('chd,kd->chk', Wk.T.reshape(C, H, D), Wdk).reshape(C, H * K)

    aq, bkT_hi, bkT_lo = pl.pallas_call(
        _digit_kernel,
        out_shape=(
            jax.ShapeDtypeStruct((H, T, K), jnp.float32),
            jax.ShapeDtypeStruct((H, K, T), jnp.float32),
            jax.ShapeDtypeStruct((H, K, T), jnp.float32),
        ),
    )(x2, weq, wek)

    v = pl.pallas_call(
        _v_kernel,
        grid=(H,),
        in_specs=[
            pl.BlockSpec((T, C), lambda h: (0, 0)),
            pl.BlockSpec((1, C, D), lambda h: (h, 0, 0)),
        ],
        out_specs=pl.BlockSpec((1, T, VP), lambda h: (h, 0, 0)),
        out_shape=jax.ShapeDtypeStruct((H, T, VP), jnp.float32),
        compiler_params=pltpu.CompilerParams(
            dimension_semantics=("arbitrary",),
        ),
    )(x2, Wv.T.reshape(C, H, D).transpose(1, 0, 2))

    mask = jnp.tril(jnp.ones((TQ, TQ), jnp.float32))

    nq = T // TQ
    y = pl.pallas_call(
        _attn_kernel,
        grid=(nq,),
        in_specs=[
            pl.BlockSpec((H, TQ, K), lambda i: (0, i, 0)),
            pl.BlockSpec((H, K, T), lambda i: (0, 0, 0)),
            pl.BlockSpec((H, K, T), lambda i: (0, 0, 0)),
            pl.BlockSpec((H, T, VP), lambda i: (0, 0, 0)),
            pl.BlockSpec((TQ, TQ), lambda i: (0, 0)),
            pl.BlockSpec((H * D, C), lambda i: (0, 0)),
        ],
        out_specs=pl.BlockSpec((TQ, C), lambda i: (i, 0)),
        out_shape=jax.ShapeDtypeStruct((T, C), jnp.float32),
        compiler_params=pltpu.CompilerParams(
            dimension_semantics=("arbitrary",),
        ),
    )(aq, bkT_hi, bkT_lo, v, mask, Wo.T)
    return y.reshape(B, T, C)


def kernel(x, Wq, Wk, Wv, Wo, Wdq, Wdk):
    return _forward(x, Wq, Wk, Wv, Wo, Wdq, Wdk)
